# baseline (speedup 1.0000x reference)
import functools

import jax
import jax.numpy as jnp
from jax import lax
from jax.experimental import pallas as pl
from jax.experimental.pallas import tpu as pltpu
from jax.experimental.pallas import tpu_sc as plsc

B, S, D = 2, 2048, 1024
E, K = 8, 2
N = B * S
CAP = N
R = E * CAP
TILE = 256
T_MAX = N * K // TILE + E
TT = 128
NEG = -3.0e38


def _gating_body(x_ref, wg_ref, pos1_ref, pos2_ref, w1_ref, w2_ref,
                 cnt_ref, carry_ref):
    step = pl.program_id(0)

    @pl.when(step == 0)
    def _init():
        carry_ref[...] = jnp.zeros((128, 128), jnp.float32)

    lg = lax.dot_general(wg_ref[...], x_ref[...],
                         (((1,), (1,)), ((), ())))
    e_iota = lax.broadcasted_iota(jnp.int32, (128, 128), 0)
    lg = jnp.where(e_iota < E, lg, NEG)

    m1 = jnp.max(lg, axis=0, keepdims=True)
    i1 = jnp.min(jnp.where(lg == m1, e_iota, E), axis=0, keepdims=True)
    lg2 = jnp.where(e_iota == i1, NEG, lg)
    m2 = jnp.max(lg2, axis=0, keepdims=True)
    i2 = jnp.min(jnp.where(lg2 == m2, e_iota, E), axis=0, keepdims=True)

    s = jnp.exp(m2 - m1)
    w1 = 1.0 / (1.0 + s)
    w2 = s * w1

    oh = ((e_iota == i1) | (e_iota == i2)).astype(jnp.float32)
    t_iota_c = lax.broadcasted_iota(jnp.int32, (128, 128), 1)
    ustrict = (e_iota < t_iota_c).astype(jnp.float32)
    excl = lax.dot_general(oh, ustrict, (((1,), (0,)), ((), ())))

    carry = carry_ref[...]
    ranks = excl + carry
    rank1 = jnp.sum(jnp.where(e_iota == i1, ranks, 0.0), axis=0,
                    keepdims=True)
    rank2 = jnp.sum(jnp.where(e_iota == i2, ranks, 0.0), axis=0,
                    keepdims=True)
    pos1 = (i1 * CAP + rank1.astype(jnp.int32))
    pos2 = (i2 * CAP + rank2.astype(jnp.int32))

    pos1_ref[...] = pos1.reshape(1, 1, 128)
    pos2_ref[...] = pos2.reshape(1, 1, 128)
    w1_ref[...] = w1.reshape(1, 1, 128)
    w2_ref[...] = w2.reshape(1, 1, 128)

    tot = jnp.sum(oh, axis=1, keepdims=True)
    carry_new = carry + jnp.broadcast_to(tot, (128, 128))
    carry_ref[...] = carry_new

    @pl.when(step == N // TT - 1)
    def _emit_counts():
        cnt_ref[...] = carry_new[0:8, :]


def _gating(x2d, wg_pad):
    grid = (N // TT,)
    return pl.pallas_call(
        _gating_body,
        grid=grid,
        in_specs=[
            pl.BlockSpec((TT, D), lambda i: (i, 0)),
            pl.BlockSpec((128, D), lambda i: (0, 0)),
        ],
        out_specs=[
            pl.BlockSpec((1, 1, 128), lambda i: (i, 0, 0)),
            pl.BlockSpec((1, 1, 128), lambda i: (i, 0, 0)),
            pl.BlockSpec((1, 1, 128), lambda i: (i, 0, 0)),
            pl.BlockSpec((1, 1, 128), lambda i: (i, 0, 0)),
            pl.BlockSpec((8, 128), lambda i: (0, 0)),
        ],
        out_shape=[
            jax.ShapeDtypeStruct((N // TT, 1, TT), jnp.int32),
            jax.ShapeDtypeStruct((N // TT, 1, TT), jnp.int32),
            jax.ShapeDtypeStruct((N // TT, 1, TT), jnp.float32),
            jax.ShapeDtypeStruct((N // TT, 1, TT), jnp.float32),
            jax.ShapeDtypeStruct((8, 128), jnp.float32),
        ],
        scratch_shapes=[pltpu.VMEM((128, 128), jnp.float32)],
    )(x2d, wg_pad)


def _scatter_kernel(pos1_hbm, pos2_hbm, x_hbm, xs_hbm,
                    idx1_v, idx2_v, xbuf, sem1, sem2):
    wid = lax.axis_index("s") * 2 + lax.axis_index("c")
    base = wid * 128
    pltpu.sync_copy(pos1_hbm.at[pl.ds(wid * 4, 4)], idx1_v)
    pltpu.sync_copy(pos2_hbm.at[pl.ds(wid * 4, 4)], idx2_v)
    for c in range(4):
        pltpu.sync_copy(x_hbm.at[pl.ds(base + c * 32, 32)], xbuf)
        h1 = pltpu.make_async_copy(xbuf, xs_hbm.at[idx1_v.at[c]], sem1)
        h2 = pltpu.make_async_copy(xbuf, xs_hbm.at[idx2_v.at[c]], sem2)
        h1.start()
        h2.start()
        h1.wait()
        h2.wait()


def _scatter_x(pos1_2d, pos2_2d, x2d):
    mesh = plsc.VectorSubcoreMesh(core_axis_name="c", subcore_axis_name="s")
    fn = functools.partial(
        pl.kernel,
        mesh=mesh,
        out_type=jax.ShapeDtypeStruct((R, D), jnp.float32),
        scratch_types=[
            pltpu.VMEM((4, 32), jnp.int32),
            pltpu.VMEM((4, 32), jnp.int32),
            pltpu.VMEM((32, D), jnp.float32),
            pltpu.SemaphoreType.DMA,
            pltpu.SemaphoreType.DMA,
        ],
    )(_scatter_kernel)
    return fn(pos1_2d, pos2_2d, x2d)


def _mm_body(et_ref, rt_ref, x_ref, w_ref, y_ref):
    xb = x_ref[...].astype(jnp.bfloat16)
    wb = w_ref[0].astype(jnp.bfloat16)
    y_ref[...] = lax.dot_general(
        xb, wb, (((1,), (1,)), ((), ())),
        preferred_element_type=jnp.float32)


def _grouped_matmul(xs, wexp, e_t, r_t):
    grid_spec = pltpu.PrefetchScalarGridSpec(
        num_scalar_prefetch=2,
        grid=(T_MAX,),
        in_specs=[
            pl.BlockSpec((TILE, D),
                         lambda j, et, rt: (et[j] * (CAP // TILE) + rt[j], 0)),
            pl.BlockSpec((1, D, D), lambda j, et, rt: (et[j], 0, 0)),
        ],
        out_specs=pl.BlockSpec(
            (TILE, D), lambda j, et, rt: (et[j] * (CAP // TILE) + rt[j], 0)),
    )
    return pl.pallas_call(
        _mm_body,
        grid_spec=grid_spec,
        out_shape=jax.ShapeDtypeStruct((R, D), jnp.float32),
    )(e_t, r_t, xs, wexp)


def _combine_kernel(pos1_hbm, pos2_hbm, w1_hbm, w2_hbm, y_hbm, out_hbm,
                    idx1_v, idx2_v, w1_v, w2_v, y1buf, y2buf, obuf,
                    sem1, sem2):
    wid = lax.axis_index("s") * 2 + lax.axis_index("c")
    base = wid * 128
    pltpu.sync_copy(pos1_hbm.at[pl.ds(wid * 4, 4)], idx1_v)
    pltpu.sync_copy(pos2_hbm.at[pl.ds(wid * 4, 4)], idx2_v)
    pltpu.sync_copy(w1_hbm.at[pl.ds(base, 128)], w1_v)
    pltpu.sync_copy(w2_hbm.at[pl.ds(base, 128)], w2_v)
    for c in range(4):
        h1 = pltpu.make_async_copy(y_hbm.at[idx1_v.at[c]], y1buf, sem1)
        h2 = pltpu.make_async_copy(y_hbm.at[idx2_v.at[c]], y2buf, sem2)
        h1.start()
        h2.start()
        h1.wait()
        h2.wait()

        for g in range(2):
            w1blk = w1_v[pl.ds(c * 32 + g * 16, 16)]
            w2blk = w2_v[pl.ds(c * 32 + g * 16, 16)]

            dnums = lax.GatherDimensionNumbers(
                offset_dims=(), collapsed_slice_dims=(0,),
                start_index_map=(0,))

            def row_body(r, _):
                lane = jnp.full((16, 1), r, jnp.int32)
                w1s = lax.gather(w1blk, lane, dnums, (1,),
                                 mode=lax.GatherScatterMode.PROMISE_IN_BOUNDS)
                w2s = lax.gather(w2blk, lane, dnums, (1,),
                                 mode=lax.GatherScatterMode.PROMISE_IN_BOUNDS)
                row = g * 16 + r

                def f_body(f, _):
                    sl = pl.ds(f * 16, 16)
                    obuf[row, sl] = (y1buf[row, sl] * w1s
                                     + y2buf[row, sl] * w2s)
                    return 0

                lax.fori_loop(0, D // 16, f_body, 0)
                return 0

            lax.fori_loop(0, 16, row_body, 0)
        pltpu.sync_copy(obuf, out_hbm.at[pl.ds(base + c * 32, 32)])


def _combine(pos1_2d, pos2_2d, w1, w2, y):
    mesh = plsc.VectorSubcoreMesh(core_axis_name="c", subcore_axis_name="s")
    fn = functools.partial(
        pl.kernel,
        mesh=mesh,
        out_type=jax.ShapeDtypeStruct((N, D), jnp.float32),
        scratch_types=[
            pltpu.VMEM((4, 32), jnp.int32),
            pltpu.VMEM((4, 32), jnp.int32),
            pltpu.VMEM((128,), jnp.float32),
            pltpu.VMEM((128,), jnp.float32),
            pltpu.VMEM((32, D), jnp.float32),
            pltpu.VMEM((32, D), jnp.float32),
            pltpu.VMEM((32, D), jnp.float32),
            pltpu.SemaphoreType.DMA,
            pltpu.SemaphoreType.DMA,
        ],
    )(_combine_kernel)
    return fn(pos1_2d, pos2_2d, w1, w2, y)


def kernel(x, Wg, Wexp):
    x2d = x.reshape(N, D)
    wg_pad = jnp.zeros((128, D), jnp.float32).at[:E].set(Wg)

    pos1, pos2, w1, w2, cnt = _gating(x2d, wg_pad)
    pos1_2d = pos1.reshape(N // 32, 32)
    pos2_2d = pos2.reshape(N // 32, 32)
    w1 = w1.reshape(N)
    w2 = w2.reshape(N)

    counts = cnt[:, 0].astype(jnp.int32)
    nt = (counts + TILE - 1) // TILE
    e_rep = jnp.repeat(jnp.arange(E, dtype=jnp.int32), nt,
                       total_repeat_length=T_MAX)
    starts = jnp.concatenate([jnp.zeros((1,), jnp.int32),
                              jnp.cumsum(nt)[:-1].astype(jnp.int32)])
    j_iota = jnp.arange(T_MAX, dtype=jnp.int32)
    valid = j_iota < jnp.sum(nt)
    e_t = jnp.where(valid, e_rep, 0)
    r_t = jnp.where(valid, j_iota - starts[e_rep], 0)

    xs = _scatter_x(pos1_2d, pos2_2d, x2d)
    y = _grouped_matmul(xs, Wexp, e_t, r_t)
    out = _combine(pos1_2d, pos2_2d, w1, w2, y)
    return out.reshape(B, S, D)


# speedup vs baseline: 1.1174x; 1.1174x over previous
import functools

import jax
import jax.numpy as jnp
from jax import lax
from jax.experimental import pallas as pl
from jax.experimental.pallas import tpu as pltpu
from jax.experimental.pallas import tpu_sc as plsc

B, S, D = 2, 2048, 1024
E, K = 8, 2
N = B * S
CAP = N
R = E * CAP
TILE = 256
T_MAX = N * K // TILE + E
TT = 128
NEG = -3.0e38


def _gating_body(x_ref, wg_ref, wexp_ref, pos1_ref, pos2_ref, w1_ref, w2_ref,
                 cnt_ref, wbf_ref, carry_ref):
    step = pl.program_id(0)
    wbf_ref[...] = wexp_ref[...].astype(jnp.bfloat16)

    @pl.when(step == 0)
    def _init():
        carry_ref[...] = jnp.zeros((128, 128), jnp.float32)

    lg = lax.dot_general(wg_ref[...], x_ref[...],
                         (((1,), (1,)), ((), ())))
    e_iota = lax.broadcasted_iota(jnp.int32, (128, 128), 0)
    lg = jnp.where(e_iota < E, lg, NEG)

    m1 = jnp.max(lg, axis=0, keepdims=True)
    i1 = jnp.min(jnp.where(lg == m1, e_iota, E), axis=0, keepdims=True)
    lg2 = jnp.where(e_iota == i1, NEG, lg)
    m2 = jnp.max(lg2, axis=0, keepdims=True)
    i2 = jnp.min(jnp.where(lg2 == m2, e_iota, E), axis=0, keepdims=True)

    s = jnp.exp(m2 - m1)
    w1 = 1.0 / (1.0 + s)
    w2 = s * w1

    oh = ((e_iota == i1) | (e_iota == i2)).astype(jnp.float32)
    t_iota_c = lax.broadcasted_iota(jnp.int32, (128, 128), 1)
    ustrict = (e_iota < t_iota_c).astype(jnp.float32)
    excl = lax.dot_general(oh, ustrict, (((1,), (0,)), ((), ())))

    carry = carry_ref[...]
    ranks = excl + carry
    rank1 = jnp.sum(jnp.where(e_iota == i1, ranks, 0.0), axis=0,
                    keepdims=True)
    rank2 = jnp.sum(jnp.where(e_iota == i2, ranks, 0.0), axis=0,
                    keepdims=True)
    pos1 = (i1 * CAP + rank1.astype(jnp.int32))
    pos2 = (i2 * CAP + rank2.astype(jnp.int32))

    pos1_ref[...] = pos1.reshape(1, 1, 128)
    pos2_ref[...] = pos2.reshape(1, 1, 128)
    w1_ref[...] = w1.reshape(1, 1, 128)
    w2_ref[...] = w2.reshape(1, 1, 128)

    tot = jnp.sum(oh, axis=1, keepdims=True)
    carry_new = carry + jnp.broadcast_to(tot, (128, 128))
    carry_ref[...] = carry_new

    @pl.when(step == N // TT - 1)
    def _emit_counts():
        cnt_ref[...] = carry_new[0:8, :]


def _gating(x2d, wg_pad, wexp):
    grid = (N // TT,)
    wchunk = D // (N // TT // E)
    return pl.pallas_call(
        _gating_body,
        grid=grid,
        in_specs=[
            pl.BlockSpec((TT, D), lambda i: (i, 0)),
            pl.BlockSpec((128, D), lambda i: (0, 0)),
            pl.BlockSpec((1, wchunk, D), lambda i: (i // 4, i % 4, 0)),
        ],
        out_specs=[
            pl.BlockSpec((1, 1, 128), lambda i: (i, 0, 0)),
            pl.BlockSpec((1, 1, 128), lambda i: (i, 0, 0)),
            pl.BlockSpec((1, 1, 128), lambda i: (i, 0, 0)),
            pl.BlockSpec((1, 1, 128), lambda i: (i, 0, 0)),
            pl.BlockSpec((8, 128), lambda i: (0, 0)),
            pl.BlockSpec((1, wchunk, D), lambda i: (i // 4, i % 4, 0)),
        ],
        out_shape=[
            jax.ShapeDtypeStruct((N // TT, 1, TT), jnp.int32),
            jax.ShapeDtypeStruct((N // TT, 1, TT), jnp.int32),
            jax.ShapeDtypeStruct((N // TT, 1, TT), jnp.float32),
            jax.ShapeDtypeStruct((N // TT, 1, TT), jnp.float32),
            jax.ShapeDtypeStruct((8, 128), jnp.float32),
            jax.ShapeDtypeStruct((E, D, D), jnp.bfloat16),
        ],
        scratch_shapes=[pltpu.VMEM((128, 128), jnp.float32)],
    )(x2d, wg_pad, wexp)


def _scatter_kernel(pos1_hbm, pos2_hbm, x_hbm, xs_hbm,
                    idx1_v, idx2_v, xba, xbb,
                    s1a, s2a, s1b, s2b, sla, slb):
    wid = lax.axis_index("s") * 2 + lax.axis_index("c")
    base = wid * 128
    pltpu.sync_copy(pos1_hbm.at[pl.ds(wid * 4, 4)], idx1_v)
    pltpu.sync_copy(pos2_hbm.at[pl.ds(wid * 4, 4)], idx2_v)

    bufs = [(xba, s1a, s2a, sla), (xbb, s1b, s2b, slb)]

    def load(c):
        xbuf, _, _, sl = bufs[c % 2]
        h = pltpu.make_async_copy(x_hbm.at[pl.ds(base + c * 32, 32)], xbuf, sl)
        h.start()
        return h

    lh = {0: load(0)}
    scat_pending = [None, None]
    for c in range(4):
        if c < 3:
            nxt = (c + 1) % 2
            if scat_pending[nxt] is not None:
                scat_pending[nxt][0].wait()
                scat_pending[nxt][1].wait()
                scat_pending[nxt] = None
            lh[c + 1] = load(c + 1)
        lh[c].wait()
        xbuf, s1, s2, _ = bufs[c % 2]
        h1 = pltpu.make_async_copy(xbuf, xs_hbm.at[idx1_v.at[c]], s1)
        h2 = pltpu.make_async_copy(xbuf, xs_hbm.at[idx2_v.at[c]], s2)
        h1.start()
        h2.start()
        scat_pending[c % 2] = (h1, h2)
    for p in scat_pending:
        if p is not None:
            p[0].wait()
            p[1].wait()


def _scatter_x(pos1_2d, pos2_2d, x2d):
    mesh = plsc.VectorSubcoreMesh(core_axis_name="c", subcore_axis_name="s")
    fn = functools.partial(
        pl.kernel,
        mesh=mesh,
        out_type=jax.ShapeDtypeStruct((R, D), jnp.float32),
        scratch_types=[
            pltpu.VMEM((4, 32), jnp.int32),
            pltpu.VMEM((4, 32), jnp.int32),
            pltpu.VMEM((32, D), jnp.float32),
            pltpu.VMEM((32, D), jnp.float32),
            pltpu.SemaphoreType.DMA,
            pltpu.SemaphoreType.DMA,
            pltpu.SemaphoreType.DMA,
            pltpu.SemaphoreType.DMA,
            pltpu.SemaphoreType.DMA,
            pltpu.SemaphoreType.DMA,
        ],
    )(_scatter_kernel)
    return fn(pos1_2d, pos2_2d, x2d)


def _mm_body(et_ref, rt_ref, x_ref, w_ref, y_ref):
    xb = x_ref[...].astype(jnp.bfloat16)
    y_ref[...] = lax.dot_general(
        xb, w_ref[0], (((1,), (1,)), ((), ())),
        preferred_element_type=jnp.float32)


def _grouped_matmul(xs, wexp, e_t, r_t):
    grid_spec = pltpu.PrefetchScalarGridSpec(
        num_scalar_prefetch=2,
        grid=(T_MAX,),
        in_specs=[
            pl.BlockSpec((TILE, D),
                         lambda j, et, rt: (et[j] * (CAP // TILE) + rt[j], 0)),
            pl.BlockSpec((1, D, D), lambda j, et, rt: (et[j], 0, 0)),
        ],
        out_specs=pl.BlockSpec(
            (TILE, D), lambda j, et, rt: (et[j] * (CAP // TILE) + rt[j], 0)),
    )
    return pl.pallas_call(
        _mm_body,
        grid_spec=grid_spec,
        out_shape=jax.ShapeDtypeStruct((R, D), jnp.float32),
    )(e_t, r_t, xs, wexp)


def _combine_kernel(pos1_hbm, pos2_hbm, w1_hbm, w2_hbm, y_hbm, out_hbm,
                    idx1_v, idx2_v, w1_v, w2_v,
                    y1a, y2a, oa, y1b, y2b, ob,
                    sem1a, sem2a, semoa, sem1b, sem2b, semob):
    wid = lax.axis_index("s") * 2 + lax.axis_index("c")
    base = wid * 128
    pltpu.sync_copy(pos1_hbm.at[pl.ds(wid * 8, 8)], idx1_v)
    pltpu.sync_copy(pos2_hbm.at[pl.ds(wid * 8, 8)], idx2_v)
    pltpu.sync_copy(w1_hbm.at[pl.ds(base, 128)], w1_v)
    pltpu.sync_copy(w2_hbm.at[pl.ds(base, 128)], w2_v)

    bufs = [(y1a, y2a, oa, sem1a, sem2a, semoa),
            (y1b, y2b, ob, sem1b, sem2b, semob)]

    def gathers(c):
        y1buf, y2buf, _, s1, s2, _ = bufs[c % 2]
        h1 = pltpu.make_async_copy(y_hbm.at[idx1_v.at[c]], y1buf, s1)
        h2 = pltpu.make_async_copy(y_hbm.at[idx2_v.at[c]], y2buf, s2)
        h1.start()
        h2.start()
        return h1, h2

    dnums = lax.GatherDimensionNumbers(
        offset_dims=(), collapsed_slice_dims=(0,), start_index_map=(0,))

    hs = {0: gathers(0)}
    out_pending = [None, None]
    for c in range(8):
        if c < 7:
            hs[c + 1] = gathers(c + 1)
        hs[c][0].wait()
        hs[c][1].wait()
        y1buf, y2buf, obuf, _, _, so = bufs[c % 2]
        if out_pending[c % 2] is not None:
            out_pending[c % 2].wait()
        w1blk = w1_v[pl.ds(c * 16, 16)]
        w2blk = w2_v[pl.ds(c * 16, 16)]

        def row_body(r, _):
            lane = jnp.full((16, 1), r, jnp.int32)
            w1s = lax.gather(w1blk, lane, dnums, (1,),
                             mode=lax.GatherScatterMode.PROMISE_IN_BOUNDS)
            w2s = lax.gather(w2blk, lane, dnums, (1,),
                             mode=lax.GatherScatterMode.PROMISE_IN_BOUNDS)
            for f in range(D // 16):
                sl = pl.ds(f * 16, 16)
                obuf[r, sl] = y1buf[r, sl] * w1s + y2buf[r, sl] * w2s
            return 0

        lax.fori_loop(0, 16, row_body, 0)
        oh = pltpu.make_async_copy(obuf, out_hbm.at[pl.ds(base + c * 16, 16)],
                                   so)
        oh.start()
        out_pending[c % 2] = oh
    out_pending[0].wait()
    out_pending[1].wait()


def _combine(pos1_e, pos2_e, w1, w2, y):
    mesh = plsc.VectorSubcoreMesh(core_axis_name="c", subcore_axis_name="s")
    fn = functools.partial(
        pl.kernel,
        mesh=mesh,
        out_type=jax.ShapeDtypeStruct((N, D), jnp.float32),
        scratch_types=[
            pltpu.VMEM((8, 16), jnp.int32),
            pltpu.VMEM((8, 16), jnp.int32),
            pltpu.VMEM((128,), jnp.float32),
            pltpu.VMEM((128,), jnp.float32),
            pltpu.VMEM((16, D), jnp.float32),
            pltpu.VMEM((16, D), jnp.float32),
            pltpu.VMEM((16, D), jnp.float32),
            pltpu.VMEM((16, D), jnp.float32),
            pltpu.VMEM((16, D), jnp.float32),
            pltpu.VMEM((16, D), jnp.float32),
            pltpu.SemaphoreType.DMA,
            pltpu.SemaphoreType.DMA,
            pltpu.SemaphoreType.DMA,
            pltpu.SemaphoreType.DMA,
            pltpu.SemaphoreType.DMA,
            pltpu.SemaphoreType.DMA,
        ],
    )(_combine_kernel)
    return fn(pos1_e, pos2_e, w1, w2, y)


def kernel(x, Wg, Wexp):
    x2d = x.reshape(N, D)
    wg_pad = jnp.zeros((128, D), jnp.float32).at[:E].set(Wg)

    pos1, pos2, w1, w2, cnt, wbf = _gating(x2d, wg_pad, Wexp)
    pos1_2d = pos1.reshape(N // 32, 32)
    pos2_2d = pos2.reshape(N // 32, 32)
    w1 = w1.reshape(N)
    w2 = w2.reshape(N)

    counts = cnt[:, 0].astype(jnp.int32)
    nt = (counts + TILE - 1) // TILE
    e_rep = jnp.repeat(jnp.arange(E, dtype=jnp.int32), nt,
                       total_repeat_length=T_MAX)
    starts = jnp.concatenate([jnp.zeros((1,), jnp.int32),
                              jnp.cumsum(nt)[:-1].astype(jnp.int32)])
    j_iota = jnp.arange(T_MAX, dtype=jnp.int32)
    valid = j_iota < jnp.sum(nt)
    e_t = jnp.where(valid, e_rep, 0)
    r_t = jnp.where(valid, j_iota - starts[e_rep], 0)

    xs = _scatter_x(pos1_2d, pos2_2d, x2d)
    y = _grouped_matmul(xs, wbf, e_t, r_t)
    out = _combine(pos1.reshape(N // 16, 16), pos2.reshape(N // 16, 16),
                   w1, w2, y)
    return out.reshape(B, S, D)


# speedup vs baseline: 1.2903x; 1.1547x over previous
import functools

import jax
import jax.numpy as jnp
from jax import lax
from jax.experimental import pallas as pl
from jax.experimental.pallas import tpu as pltpu
from jax.experimental.pallas import tpu_sc as plsc

B, S, D = 2, 2048, 1024
E, K = 8, 2
N = B * S
CAP = N
R = E * CAP
TILE = 512
T_MAX = N * K // TILE + E
TT = 256
NSTEP = N // TT


def _gating_body(x_ref, wg_ref, wexp_ref, pos1_ref, pos2_ref, w1_ref, w2_ref,
                 et_ref, rt_ref, wbf_ref, carry_ref):
    step = pl.program_id(0)
    wbf_ref[...] = wexp_ref[...].astype(jnp.bfloat16)

    @pl.when(step == 0)
    def _init():
        carry_ref[...] = jnp.zeros((E, TT), jnp.float32)

    lg = lax.dot_general(wg_ref[...], x_ref[...],
                         (((1,), (1,)), ((), ())))
    e_iota = lax.broadcasted_iota(jnp.int32, (E, TT), 0)

    m1 = jnp.max(lg, axis=0, keepdims=True)
    i1 = jnp.min(jnp.where(lg == m1, e_iota, E), axis=0, keepdims=True)
    lg2 = jnp.where(e_iota == i1, -3.0e38, lg)
    m2 = jnp.max(lg2, axis=0, keepdims=True)
    i2 = jnp.min(jnp.where(lg2 == m2, e_iota, E), axis=0, keepdims=True)

    s = jnp.exp(m2 - m1)
    w1 = 1.0 / (1.0 + s)
    w2 = s * w1

    oh = ((e_iota == i1) | (e_iota == i2)).astype(jnp.float32)
    t_iota_r = lax.broadcasted_iota(jnp.int32, (TT, TT), 0)
    t_iota_c = lax.broadcasted_iota(jnp.int32, (TT, TT), 1)
    ustrict = (t_iota_r < t_iota_c).astype(jnp.float32)
    excl = lax.dot_general(oh, ustrict, (((1,), (0,)), ((), ())))

    carry = carry_ref[...]
    ranks = excl + carry
    rank1 = jnp.sum(jnp.where(e_iota == i1, ranks, 0.0), axis=0,
                    keepdims=True)
    rank2 = jnp.sum(jnp.where(e_iota == i2, ranks, 0.0), axis=0,
                    keepdims=True)
    pos1 = (i1 * CAP + rank1.astype(jnp.int32))
    pos2 = (i2 * CAP + rank2.astype(jnp.int32))

    pos1_ref[...] = pos1.reshape(1, 1, TT)
    pos2_ref[...] = pos2.reshape(1, 1, TT)
    w1_ref[...] = w1.reshape(1, 1, TT)
    w2_ref[...] = w2.reshape(1, 1, TT)

    tot = jnp.sum(oh, axis=1, keepdims=True)
    carry_new = carry + jnp.broadcast_to(tot, (E, TT))
    carry_ref[...] = carry_new

    @pl.when(step == NSTEP - 1)
    def _emit_tile_map():
        nt = jnp.floor((carry_new + (TILE - 1)) * (1.0 / TILE))
        tri = (lax.broadcasted_iota(jnp.int32, (E, E), 0)
               >= lax.broadcasted_iota(jnp.int32, (E, E), 1)).astype(
                   jnp.float32)
        cum = lax.dot_general(tri, nt, (((1,), (0,)), ((), ())))
        j_lane = lax.broadcasted_iota(jnp.int32, (E, TT), 1).astype(
            jnp.float32)
        ge = (cum <= j_lane).astype(jnp.float32)
        e_t = jnp.sum(ge, axis=0, keepdims=True)
        start = cum - nt
        e_t_b = jnp.broadcast_to(e_t, (E, TT))
        e_iota_f = e_iota.astype(jnp.float32)
        st_g = jnp.sum(jnp.where(e_iota_f == e_t_b, start, 0.0), axis=0,
                       keepdims=True)
        j1 = lax.broadcasted_iota(jnp.int32, (1, TT), 1).astype(jnp.float32)
        total = cum[E - 1:E, :]
        valid = j1 < total
        et_ref[...] = jnp.where(valid, e_t, 0.0).astype(jnp.int32)
        rt_ref[...] = jnp.where(valid, j1 - st_g, 0.0).astype(jnp.int32)


def _gating(x2d, wg, wexp):
    wchunk = D // (NSTEP // E)
    return pl.pallas_call(
        _gating_body,
        grid=(NSTEP,),
        in_specs=[
            pl.BlockSpec((TT, D), lambda i: (i, 0)),
            pl.BlockSpec((E, D), lambda i: (0, 0)),
            pl.BlockSpec((1, wchunk, D), lambda i: (i // 2, i % 2, 0)),
        ],
        out_specs=[
            pl.BlockSpec((1, 1, TT), lambda i: (i, 0, 0)),
            pl.BlockSpec((1, 1, TT), lambda i: (i, 0, 0)),
            pl.BlockSpec((1, 1, TT), lambda i: (i, 0, 0)),
            pl.BlockSpec((1, 1, TT), lambda i: (i, 0, 0)),
            pl.BlockSpec((1, TT), lambda i: (0, 0)),
            pl.BlockSpec((1, TT), lambda i: (0, 0)),
            pl.BlockSpec((1, wchunk, D), lambda i: (i // 2, i % 2, 0)),
        ],
        out_shape=[
            jax.ShapeDtypeStruct((NSTEP, 1, TT), jnp.int32),
            jax.ShapeDtypeStruct((NSTEP, 1, TT), jnp.int32),
            jax.ShapeDtypeStruct((NSTEP, 1, TT), jnp.float32),
            jax.ShapeDtypeStruct((NSTEP, 1, TT), jnp.float32),
            jax.ShapeDtypeStruct((1, TT), jnp.int32),
            jax.ShapeDtypeStruct((1, TT), jnp.int32),
            jax.ShapeDtypeStruct((E, D, D), jnp.bfloat16),
        ],
        scratch_shapes=[pltpu.VMEM((E, TT), jnp.float32)],
    )(x2d, wg, wexp)


def _scatter_kernel(pos1_hbm, pos2_hbm, x_hbm, xs_hbm,
                    idx1_v, idx2_v, xba, xbb,
                    s1a, s2a, s1b, s2b, sla, slb):
    wid = lax.axis_index("s") * 2 + lax.axis_index("c")
    base = wid * 128
    pltpu.sync_copy(pos1_hbm.at[pl.ds(wid * 4, 4)], idx1_v)
    pltpu.sync_copy(pos2_hbm.at[pl.ds(wid * 4, 4)], idx2_v)

    bufs = [(xba, s1a, s2a, sla), (xbb, s1b, s2b, slb)]

    def load(c):
        xbuf, _, _, sl = bufs[c % 2]
        h = pltpu.make_async_copy(x_hbm.at[pl.ds(base + c * 32, 32)], xbuf, sl)
        h.start()
        return h

    lh = {0: load(0)}
    scat_pending = [None, None]
    for c in range(4):
        if c < 3:
            nxt = (c + 1) % 2
            if scat_pending[nxt] is not None:
                scat_pending[nxt][0].wait()
                scat_pending[nxt][1].wait()
                scat_pending[nxt] = None
            lh[c + 1] = load(c + 1)
        lh[c].wait()
        xbuf, s1, s2, _ = bufs[c % 2]
        h1 = pltpu.make_async_copy(xbuf, xs_hbm.at[idx1_v.at[c]], s1)
        h2 = pltpu.make_async_copy(xbuf, xs_hbm.at[idx2_v.at[c]], s2)
        h1.start()
        h2.start()
        scat_pending[c % 2] = (h1, h2)
    for p in scat_pending:
        if p is not None:
            p[0].wait()
            p[1].wait()


def _scatter_x(pos1_2d, pos2_2d, x2d):
    mesh = plsc.VectorSubcoreMesh(core_axis_name="c", subcore_axis_name="s")
    fn = functools.partial(
        pl.kernel,
        mesh=mesh,
        out_type=jax.ShapeDtypeStruct((R, D), jnp.float32),
        scratch_types=[
            pltpu.VMEM((4, 32), jnp.int32),
            pltpu.VMEM((4, 32), jnp.int32),
            pltpu.VMEM((32, D), jnp.float32),
            pltpu.VMEM((32, D), jnp.float32),
            pltpu.SemaphoreType.DMA,
            pltpu.SemaphoreType.DMA,
            pltpu.SemaphoreType.DMA,
            pltpu.SemaphoreType.DMA,
            pltpu.SemaphoreType.DMA,
            pltpu.SemaphoreType.DMA,
        ],
    )(_scatter_kernel)
    return fn(pos1_2d, pos2_2d, x2d)


def _mm_body(et_ref, rt_ref, x_ref, w_ref, y_ref):
    xb = x_ref[...].astype(jnp.bfloat16)
    y_ref[...] = lax.dot_general(
        xb, w_ref[0], (((1,), (1,)), ((), ())),
        preferred_element_type=jnp.float32)


def _grouped_matmul(xs, wexp, e_t, r_t):
    grid_spec = pltpu.PrefetchScalarGridSpec(
        num_scalar_prefetch=2,
        grid=(T_MAX,),
        in_specs=[
            pl.BlockSpec((TILE, D),
                         lambda j, et, rt: (et[0, j] * (CAP // TILE)
                                            + rt[0, j], 0)),
            pl.BlockSpec((1, D, D), lambda j, et, rt: (et[0, j], 0, 0)),
        ],
        out_specs=pl.BlockSpec(
            (TILE, D), lambda j, et, rt: (et[0, j] * (CAP // TILE)
                                          + rt[0, j], 0)),
    )
    return pl.pallas_call(
        _mm_body,
        grid_spec=grid_spec,
        out_shape=jax.ShapeDtypeStruct((R, D), jnp.float32),
    )(e_t, r_t, xs, wexp)


def _combine_kernel(pos1_hbm, pos2_hbm, w1_hbm, w2_hbm, y_hbm, out_hbm,
                    idx1_v, idx2_v, w1_v, w2_v,
                    y1a, y2a, oa, y1b, y2b, ob,
                    sem1a, sem2a, semoa, sem1b, sem2b, semob):
    wid = lax.axis_index("s") * 2 + lax.axis_index("c")
    base = wid * 128
    pltpu.sync_copy(pos1_hbm.at[pl.ds(wid * 8, 8)], idx1_v)
    pltpu.sync_copy(pos2_hbm.at[pl.ds(wid * 8, 8)], idx2_v)
    pltpu.sync_copy(w1_hbm.at[pl.ds(base, 128)], w1_v)
    pltpu.sync_copy(w2_hbm.at[pl.ds(base, 128)], w2_v)

    bufs = [(y1a, y2a, oa, sem1a, sem2a, semoa),
            (y1b, y2b, ob, sem1b, sem2b, semob)]

    def gathers(c):
        y1buf, y2buf, _, s1, s2, _ = bufs[c % 2]
        h1 = pltpu.make_async_copy(y_hbm.at[idx1_v.at[c]], y1buf, s1)
        h2 = pltpu.make_async_copy(y_hbm.at[idx2_v.at[c]], y2buf, s2)
        h1.start()
        h2.start()
        return h1, h2

    dnums = lax.GatherDimensionNumbers(
        offset_dims=(), collapsed_slice_dims=(0,), start_index_map=(0,))

    hs = {0: gathers(0)}
    out_pending = [None, None]
    for c in range(8):
        if c < 7:
            hs[c + 1] = gathers(c + 1)
        hs[c][0].wait()
        hs[c][1].wait()
        y1buf, y2buf, obuf, _, _, so = bufs[c % 2]
        if out_pending[c % 2] is not None:
            out_pending[c % 2].wait()
        w1blk = w1_v[pl.ds(c * 16, 16)]
        w2blk = w2_v[pl.ds(c * 16, 16)]

        def row_body(r, _):
            lane = jnp.full((16, 1), r, jnp.int32)
            w1s = lax.gather(w1blk, lane, dnums, (1,),
                             mode=lax.GatherScatterMode.PROMISE_IN_BOUNDS)
            w2s = lax.gather(w2blk, lane, dnums, (1,),
                             mode=lax.GatherScatterMode.PROMISE_IN_BOUNDS)
            for f in range(D // 16):
                sl = pl.ds(f * 16, 16)
                obuf[r, sl] = y1buf[r, sl] * w1s + y2buf[r, sl] * w2s
            return 0

        lax.fori_loop(0, 16, row_body, 0)
        oh = pltpu.make_async_copy(obuf, out_hbm.at[pl.ds(base + c * 16, 16)],
                                   so)
        oh.start()
        out_pending[c % 2] = oh
    out_pending[0].wait()
    out_pending[1].wait()


def _combine(pos1_e, pos2_e, w1, w2, y):
    mesh = plsc.VectorSubcoreMesh(core_axis_name="c", subcore_axis_name="s")
    fn = functools.partial(
        pl.kernel,
        mesh=mesh,
        out_type=jax.ShapeDtypeStruct((N, D), jnp.float32),
        scratch_types=[
            pltpu.VMEM((8, 16), jnp.int32),
            pltpu.VMEM((8, 16), jnp.int32),
            pltpu.VMEM((128,), jnp.float32),
            pltpu.VMEM((128,), jnp.float32),
            pltpu.VMEM((16, D), jnp.float32),
            pltpu.VMEM((16, D), jnp.float32),
            pltpu.VMEM((16, D), jnp.float32),
            pltpu.VMEM((16, D), jnp.float32),
            pltpu.VMEM((16, D), jnp.float32),
            pltpu.VMEM((16, D), jnp.float32),
            pltpu.SemaphoreType.DMA,
            pltpu.SemaphoreType.DMA,
            pltpu.SemaphoreType.DMA,
            pltpu.SemaphoreType.DMA,
            pltpu.SemaphoreType.DMA,
            pltpu.SemaphoreType.DMA,
        ],
    )(_combine_kernel)
    return fn(pos1_e, pos2_e, w1, w2, y)


def kernel(x, Wg, Wexp):
    x2d = x.reshape(N, D)

    pos1, pos2, w1, w2, e_t, r_t, wbf = _gating(x2d, Wg, Wexp)
    pos1_2d = pos1.reshape(N // 32, 32)
    pos2_2d = pos2.reshape(N // 32, 32)
    w1 = w1.reshape(N)
    w2 = w2.reshape(N)

    xs = _scatter_x(pos1_2d, pos2_2d, x2d)
    y = _grouped_matmul(xs, wbf, e_t, r_t)
    out = _combine(pos1.reshape(N // 16, 16), pos2.reshape(N // 16, 16),
                   w1, w2, y)
    return out.reshape(B, S, D)
